# baseline (speedup 1.0000x reference)
import functools
from itertools import combinations

import jax
import jax.numpy as jnp
from jax.experimental import pallas as pl
from jax.experimental.pallas import tpu as pltpu

NUM_CLASSES = 7
MAX_SET_SIZE = 2

_SETS = [()]
for _sz in range(1, MAX_SET_SIZE + 1):
    _SETS.extend(combinations(range(NUM_CLASSES), _sz))
NPC = len(_SETS)
_MEMBERS = tuple(
    tuple(k for k, s in enumerate(_SETS) if c in s) for c in range(NUM_CLASSES)
)


def _body(x_hbm, o_hbm, xv, acc, ov, in_sems, out_sems):
    copies_in = [
        pltpu.make_async_copy(x_hbm.at[k], xv.at[k], in_sems.at[k])
        for k in range(NPC)
    ]
    copies_out = [
        pltpu.make_async_copy(ov.at[c], o_hbm.at[c], out_sems.at[c])
        for c in range(NUM_CLASSES)
    ]
    for c in copies_in:
        c.start()
    for k in range(NPC):
        copies_in[k].wait()
        e = jnp.exp(xv[k])
        if k == 0:
            acc[NUM_CLASSES] = e
        else:
            acc[NUM_CLASSES] = acc[NUM_CLASSES] + e
            for c in _SETS[k]:
                if k == 1 + c:
                    acc[c] = e
                else:
                    acc[c] = acc[c] + e
    inv = 1.0 / acc[NUM_CLASSES]
    for c in range(NUM_CLASSES):
        ov[c] = acc[c] * inv
        copies_out[c].start()
    for c in copies_out:
        c.wait()


@jax.jit
def kernel(powerset, mapping_matrix):
    b, f, npc = powerset.shape
    x_t = jnp.transpose(powerset, (2, 0, 1))
    out_t = pl.pallas_call(
        _body,
        in_specs=[pl.BlockSpec(memory_space=pl.ANY)],
        out_specs=pl.BlockSpec(memory_space=pl.ANY),
        out_shape=jax.ShapeDtypeStruct((NUM_CLASSES, b, f), jnp.float32),
        scratch_shapes=[
            pltpu.VMEM((NPC, b, f), jnp.float32),
            pltpu.VMEM((NUM_CLASSES + 1, b, f), jnp.float32),
            pltpu.VMEM((NUM_CLASSES, b, f), jnp.float32),
            pltpu.SemaphoreType.DMA((NPC,)),
            pltpu.SemaphoreType.DMA((NUM_CLASSES,)),
        ],
        compiler_params=pltpu.CompilerParams(
            vmem_limit_bytes=64 * 1024 * 1024,
        ),
    )(x_t)
    return jnp.transpose(out_t, (1, 2, 0))
